# speedup vs baseline: 4.6440x; 4.6440x over previous
import jax
import jax.numpy as jnp
from jax import lax
from jax.experimental import pallas as pl
from jax.experimental.pallas import tpu as pltpu
from jax.experimental.pallas import tpu_sc as plsc

N = 10000
E = 320000
D = 128
H = 256
C = 40

NC = 2
NS = 16
NW = NC * NS

K = 128
CH = -(-E // (NW * K))
EPAD = NW * CH * K
NPAD = 10240
STRIPE = NPAD // NS

CPAD = 128


def _make_deg_call():
  mesh = plsc.VectorSubcoreMesh(core_axis_name="c", subcore_axis_name="s")

  def kern(src_hbm, dst_hbm, zeros_hbm, ones_hbm,
           dego_hbm, degi_hbm,
           idx_v, ones_v, dego_sh, degi_sh):
    cid = lax.axis_index("c")
    sid = lax.axis_index("s")
    wid = cid * NS + sid

    pltpu.sync_copy(zeros_hbm, dego_sh.at[pl.ds(sid * STRIPE, STRIPE)])
    pltpu.sync_copy(zeros_hbm, degi_sh.at[pl.ds(sid * STRIPE, STRIPE)])
    pltpu.sync_copy(ones_hbm, ones_v)
    plsc.subcore_barrier()

    def body(i, carry):
      base = (wid * CH + i) * K
      pltpu.sync_copy(src_hbm.at[pl.ds(base, K)], idx_v)
      pltpu.sync_copy(ones_v, dego_sh.at[idx_v], add=True)
      pltpu.sync_copy(dst_hbm.at[pl.ds(base, K)], idx_v)
      pltpu.sync_copy(ones_v, degi_sh.at[idx_v], add=True)
      return carry

    lax.fori_loop(0, CH, body, 0)
    plsc.subcore_barrier()

    pltpu.sync_copy(dego_sh.at[pl.ds(sid * STRIPE, STRIPE)],
                    dego_hbm.at[pl.ds(cid * NPAD + sid * STRIPE, STRIPE)])
    pltpu.sync_copy(degi_sh.at[pl.ds(sid * STRIPE, STRIPE)],
                    degi_hbm.at[pl.ds(cid * NPAD + sid * STRIPE, STRIPE)])

  return pl.kernel(
      kern,
      out_type=[
          jax.ShapeDtypeStruct((NC * NPAD,), jnp.float32),
          jax.ShapeDtypeStruct((NC * NPAD,), jnp.float32),
      ],
      mesh=mesh,
      scratch_types=[
          pltpu.VMEM((K,), jnp.int32),
          pltpu.VMEM((K,), jnp.float32),
          pltpu.VMEM_SHARED((NPAD,), jnp.float32),
          pltpu.VMEM_SHARED((NPAD,), jnp.float32),
      ],
  )


def _make_agg_call():
  mesh = plsc.VectorSubcoreMesh(core_axis_name="c", subcore_axis_name="s")

  def kern(x_hbm, src_hbm, dst_hbm, zrows_hbm,
           part_hbm,
           src_v, dst_v, rows_v, acc_sh, sem):
    cid = lax.axis_index("c")
    sid = lax.axis_index("s")
    wid = cid * NS + sid

    def zbody(q, carry):
      pltpu.sync_copy(zrows_hbm, acc_sh.at[pl.ds(sid * STRIPE + q * K, K)])
      return carry
    lax.fori_loop(0, STRIPE // K, zbody, 0)
    plsc.subcore_barrier()

    def body(i, carry):
      base = (wid * CH + i) * K
      pltpu.sync_copy(src_hbm.at[pl.ds(base, K)], src_v)
      pltpu.sync_copy(dst_hbm.at[pl.ds(base, K)], dst_v)
      pltpu.async_copy(x_hbm.at[src_v], rows_v, sem).wait()
      pltpu.sync_copy(rows_v, acc_sh.at[dst_v], add=True)
      return carry

    lax.fori_loop(0, CH, body, 0)
    plsc.subcore_barrier()

    pltpu.sync_copy(acc_sh.at[pl.ds(sid * STRIPE, STRIPE)],
                    part_hbm.at[pl.ds(cid * NPAD + sid * STRIPE, STRIPE)])

  return pl.kernel(
      kern,
      out_type=jax.ShapeDtypeStruct((NC * NPAD, D), jnp.float32),
      mesh=mesh,
      scratch_types=[
          pltpu.VMEM((K,), jnp.int32),
          pltpu.VMEM((K,), jnp.int32),
          pltpu.VMEM((K, D), jnp.float32),
          pltpu.VMEM_SHARED((NPAD, D), jnp.float32),
          pltpu.SemaphoreType.DMA,
      ],
  )


def _norm_body(feat_ref, dego0_ref, dego1_ref, degi0_ref, degi1_ref,
               x_ref, normi_ref):
  deg_o = dego0_ref[...] + dego1_ref[...] + 1.0
  deg_i = degi0_ref[...] + degi1_ref[...] + 1.0
  x_ref[...] = feat_ref[...] * lax.rsqrt(deg_o)
  normi_ref[...] = lax.rsqrt(deg_i)


def _mlp_body(p0_ref, p1_ref, x_ref, normi_ref,
              w1_ref, b1_ref, w2_ref, b2_ref, out_ref):
  agg = (p0_ref[...] + p1_ref[...] + x_ref[...]) * normi_ref[...]
  h = jnp.dot(agg, w1_ref[...], preferred_element_type=jnp.float32)
  h = jnp.maximum(h + b1_ref[...], 0.0)
  lg = jnp.dot(h, w2_ref[...], preferred_element_type=jnp.float32) + b2_ref[...]
  m = jnp.max(lg, axis=1, keepdims=True)
  s = jnp.sum(jnp.exp(lg - m), axis=1, keepdims=True)
  out_ref[...] = lg - m - jnp.log(s)


def kernel(features, edge_index, W1, b1, W2, b2):
  f32 = jnp.float32
  npad_e = EPAD - E
  dummy = jnp.full((npad_e,), N, dtype=jnp.int32)
  src_p = jnp.concatenate([edge_index[0].astype(jnp.int32), dummy])
  dst_p = jnp.concatenate([edge_index[1].astype(jnp.int32), dummy])

  zeros_stripe = jnp.zeros((STRIPE,), f32)
  ones_k = jnp.ones((K,), f32)
  zrows = jnp.zeros((K, D), f32)

  dego, degi = _make_deg_call()(src_p, dst_p, zeros_stripe, ones_k)

  RB = 1000
  gridb = N // RB
  dego0 = dego[:N].reshape(N, 1)
  dego1 = dego[NPAD:NPAD + N].reshape(N, 1)
  degi0 = degi[:N].reshape(N, 1)
  degi1 = degi[NPAD:NPAD + N].reshape(N, 1)
  x_pad, norm_in = pl.pallas_call(
      _norm_body,
      grid=(gridb,),
      in_specs=[
          pl.BlockSpec((RB, D), lambda i: (i, 0)),
          pl.BlockSpec((RB, 1), lambda i: (i, 0)),
          pl.BlockSpec((RB, 1), lambda i: (i, 0)),
          pl.BlockSpec((RB, 1), lambda i: (i, 0)),
          pl.BlockSpec((RB, 1), lambda i: (i, 0)),
      ],
      out_specs=[
          pl.BlockSpec((RB, D), lambda i: (i, 0)),
          pl.BlockSpec((RB, 1), lambda i: (i, 0)),
      ],
      out_shape=[
          jax.ShapeDtypeStruct((NPAD, D), f32),
          jax.ShapeDtypeStruct((NPAD, 1), f32),
      ],
  )(features, dego0, dego1, degi0, degi1)

  part = _make_agg_call()(x_pad, src_p, dst_p, zrows)

  W2p = jnp.concatenate([W2, jnp.zeros((H, CPAD - C), f32)], axis=1)
  b2p = jnp.concatenate([b2, jnp.full((CPAD - C,), -1e30, f32)]).reshape(1, CPAD)
  b1r = b1.reshape(1, H)

  RD = 640
  gridd = NPAD // RD
  out = pl.pallas_call(
      _mlp_body,
      grid=(gridd,),
      in_specs=[
          pl.BlockSpec((RD, D), lambda i: (i, 0)),
          pl.BlockSpec((RD, D), lambda i: (i + NPAD // RD, 0)),
          pl.BlockSpec((RD, D), lambda i: (i, 0)),
          pl.BlockSpec((RD, 1), lambda i: (i, 0)),
          pl.BlockSpec((D, H), lambda i: (0, 0)),
          pl.BlockSpec((1, H), lambda i: (0, 0)),
          pl.BlockSpec((H, CPAD), lambda i: (0, 0)),
          pl.BlockSpec((1, CPAD), lambda i: (0, 0)),
      ],
      out_specs=pl.BlockSpec((RD, CPAD), lambda i: (i, 0)),
      out_shape=jax.ShapeDtypeStruct((NPAD, CPAD), f32),
  )(part, part, x_pad, norm_in, W1, b1r, W2p, b2p)

  return out[:N, :C]


# speedup vs baseline: 4.9990x; 1.0765x over previous
import jax
import jax.numpy as jnp
from jax import lax
from jax.experimental import pallas as pl
from jax.experimental.pallas import tpu as pltpu
from jax.experimental.pallas import tpu_sc as plsc

N = 10000
E = 320000
D = 128
H = 256
C = 40

NC = 2
NS = 16
NW = NC * NS

K = 128
NB = 2
CH = -(-E // (NW * K * NB)) * NB
EPAD = NW * CH * K
NPAD = 10240
STRIPE = NPAD // NS

CPAD = 128


def _make_deg_call():
  mesh = plsc.VectorSubcoreMesh(core_axis_name="c", subcore_axis_name="s")

  def kern(src_hbm, dst_hbm, zeros_hbm, ones_hbm,
           dego_hbm, degi_hbm,
           src2_v, dst2_v, ones_v, dego_sh, degi_sh,
           sa0, sa1, sa2, sa3, sb0, sb1, sb2, sb3):
    cid = lax.axis_index("c")
    sid = lax.axis_index("s")
    wid = cid * NS + sid
    sa = (sa0, sa1, sa2, sa3)
    sb = (sb0, sb1, sb2, sb3)
    NBA = 4

    pltpu.sync_copy(zeros_hbm, dego_sh.at[pl.ds(sid * STRIPE, STRIPE)])
    pltpu.sync_copy(zeros_hbm, degi_sh.at[pl.ds(sid * STRIPE, STRIPE)])
    pltpu.sync_copy(ones_hbm, ones_v)
    pltpu.sync_copy(src_hbm.at[pl.ds(wid * CH, CH)], src2_v)
    pltpu.sync_copy(dst_hbm.at[pl.ds(wid * CH, CH)], dst2_v)
    plsc.subcore_barrier()

    def body(step, carry):
      for b in range(NBA):
        i = step * NBA + b

        @pl.when(step > 0)
        def _():
          pltpu.make_async_copy(zeros_hbm.at[pl.ds(0, K)], ones_v,
                                sa[b]).wait()
          pltpu.make_async_copy(zeros_hbm.at[pl.ds(0, K)], ones_v,
                                sb[b]).wait()

        pltpu.make_async_copy(ones_v, dego_sh.at[src2_v.at[i]],
                              sa[b]).start(add=True)
        pltpu.make_async_copy(ones_v, degi_sh.at[dst2_v.at[i]],
                              sb[b]).start(add=True)
      return carry

    lax.fori_loop(0, CH // NBA, body, 0)
    for b in range(NBA):
      pltpu.make_async_copy(zeros_hbm.at[pl.ds(0, K)], ones_v,
                            sa[b]).wait()
      pltpu.make_async_copy(zeros_hbm.at[pl.ds(0, K)], ones_v,
                            sb[b]).wait()
    plsc.subcore_barrier()

    pltpu.sync_copy(dego_sh.at[pl.ds(sid * STRIPE, STRIPE)],
                    dego_hbm.at[pl.ds(cid * NPAD + sid * STRIPE, STRIPE)])
    pltpu.sync_copy(degi_sh.at[pl.ds(sid * STRIPE, STRIPE)],
                    degi_hbm.at[pl.ds(cid * NPAD + sid * STRIPE, STRIPE)])

  return pl.kernel(
      kern,
      out_type=[
          jax.ShapeDtypeStruct((NC * NPAD,), jnp.float32),
          jax.ShapeDtypeStruct((NC * NPAD,), jnp.float32),
      ],
      mesh=mesh,
      scratch_types=[
          pltpu.VMEM((CH, K), jnp.int32),
          pltpu.VMEM((CH, K), jnp.int32),
          pltpu.VMEM((K,), jnp.float32),
          pltpu.VMEM_SHARED((NPAD,), jnp.float32),
          pltpu.VMEM_SHARED((NPAD,), jnp.float32),
      ] + [pltpu.SemaphoreType.DMA] * 8,
  )


def _make_agg_call():
  mesh = plsc.VectorSubcoreMesh(core_axis_name="c", subcore_axis_name="s")
  AHEAD = 2
  HALVES = 2
  HC = CH // HALVES

  def kern(x_hbm, src_hbm, dst_hbm, zrows_hbm,
           part_hbm,
           src2_v, dst2_v, r0, r1, acc_sh,
           g0, g1, s0, s1):
    cid = lax.axis_index("c")
    sid = lax.axis_index("s")
    wid = cid * NS + sid
    gs = (g0, g1)
    ss = (s0, s1)
    rows = (r0, r1)

    @pl.when(cid == 0)
    def _():
      pltpu.sync_copy(x_hbm.at[pl.ds(sid * STRIPE, STRIPE)],
                      acc_sh.at[pl.ds(sid * STRIPE, STRIPE)])

    @pl.when(cid != 0)
    def _():
      def zbody(q, carry):
        pltpu.sync_copy(zrows_hbm, acc_sh.at[pl.ds(sid * STRIPE + q * K, K)])
        return carry
      lax.fori_loop(0, STRIPE // K, zbody, 0)

    plsc.subcore_barrier()

    for h in range(HALVES):
      pltpu.sync_copy(src_hbm.at[pl.ds(wid * CH + h * HC, HC)], src2_v)
      pltpu.sync_copy(dst_hbm.at[pl.ds(wid * CH + h * HC, HC)], dst2_v)

      for b in range(AHEAD):
        pltpu.make_async_copy(x_hbm.at[src2_v.at[b]], rows[b],
                              gs[b]).start()

      def body(step, carry):
        for b in range(NB):
          ib = step * NB + b
          pltpu.make_async_copy(x_hbm.at[pl.ds(0, K)], rows[b],
                                gs[b]).wait()
          pltpu.make_async_copy(rows[b], acc_sh.at[dst2_v.at[ib]],
                                ss[b]).start(add=True)
          jb = ib + AHEAD
          bj = (b + AHEAD) % NB

          @pl.when(jnp.logical_and(jb >= NB, jb < HC))
          def _():
            pltpu.make_async_copy(x_hbm.at[pl.ds(0, K)], rows[bj],
                                  ss[bj]).wait()

          @pl.when(jb < HC)
          def _():
            pltpu.make_async_copy(x_hbm.at[src2_v.at[jb]], rows[bj],
                                  gs[bj]).start()
        return carry

      lax.fori_loop(0, HC // NB, body, 0)
      for b in range(NB):
        pltpu.make_async_copy(x_hbm.at[pl.ds(0, K)], rows[b],
                              ss[b]).wait()

    plsc.subcore_barrier()

    pltpu.sync_copy(acc_sh.at[pl.ds(sid * STRIPE, STRIPE)],
                    part_hbm.at[pl.ds(cid * NPAD + sid * STRIPE, STRIPE)])

  return pl.kernel(
      kern,
      out_type=jax.ShapeDtypeStruct((NC * NPAD, D), jnp.float32),
      mesh=mesh,
      scratch_types=[
          pltpu.VMEM((CH // HALVES, K), jnp.int32),
          pltpu.VMEM((CH // HALVES, K), jnp.int32),
          pltpu.VMEM((K, D), jnp.float32),
          pltpu.VMEM((K, D), jnp.float32),
          pltpu.VMEM_SHARED((NPAD, D), jnp.float32),
      ] + [pltpu.SemaphoreType.DMA] * (2 * NB),
  )


def _norm_body(feat_ref, dego0_ref, dego1_ref, degi0_ref, degi1_ref,
               x_ref, normi_ref):
  deg_o = dego0_ref[...] + dego1_ref[...] + 1.0
  deg_i = degi0_ref[...] + degi1_ref[...] + 1.0
  x_ref[...] = feat_ref[...] * lax.rsqrt(deg_o)
  normi_ref[...] = lax.rsqrt(deg_i)


def _mlp_body(p0_ref, p1_ref, normi_ref,
              w1_ref, b1_ref, w2_ref, b2_ref, out_ref):
  agg = (p0_ref[...] + p1_ref[...]) * normi_ref[...]
  h = jnp.dot(agg, w1_ref[...], preferred_element_type=jnp.float32)
  h = jnp.maximum(h + b1_ref[...], 0.0)
  lg = jnp.dot(h, w2_ref[...], preferred_element_type=jnp.float32) + b2_ref[...]
  m = jnp.max(lg, axis=1, keepdims=True)
  s = jnp.sum(jnp.exp(lg - m), axis=1, keepdims=True)
  out_ref[...] = lg - m - jnp.log(s)


def kernel(features, edge_index, W1, b1, W2, b2):
  f32 = jnp.float32
  npad_e = EPAD - E
  dummy = jnp.full((npad_e,), N, dtype=jnp.int32)
  src_p = jnp.concatenate([edge_index[0].astype(jnp.int32), dummy])
  dst_p = jnp.concatenate([edge_index[1].astype(jnp.int32), dummy])
  src2 = src_p.reshape(NW * CH, K)
  dst2 = dst_p.reshape(NW * CH, K)

  zeros_stripe = jnp.zeros((STRIPE,), f32)
  ones_k = jnp.ones((K,), f32)
  zrows = jnp.zeros((K, D), f32)

  dego, degi = _make_deg_call()(src2, dst2, zeros_stripe, ones_k)

  RB = 1000
  gridb = N // RB
  dego0 = dego[:N].reshape(N, 1)
  dego1 = dego[NPAD:NPAD + N].reshape(N, 1)
  degi0 = degi[:N].reshape(N, 1)
  degi1 = degi[NPAD:NPAD + N].reshape(N, 1)
  x_pad, norm_in = pl.pallas_call(
      _norm_body,
      grid=(gridb,),
      in_specs=[
          pl.BlockSpec((RB, D), lambda i: (i, 0)),
          pl.BlockSpec((RB, 1), lambda i: (i, 0)),
          pl.BlockSpec((RB, 1), lambda i: (i, 0)),
          pl.BlockSpec((RB, 1), lambda i: (i, 0)),
          pl.BlockSpec((RB, 1), lambda i: (i, 0)),
      ],
      out_specs=[
          pl.BlockSpec((RB, D), lambda i: (i, 0)),
          pl.BlockSpec((RB, 1), lambda i: (i, 0)),
      ],
      out_shape=[
          jax.ShapeDtypeStruct((NPAD, D), f32),
          jax.ShapeDtypeStruct((NPAD, 1), f32),
      ],
  )(features, dego0, dego1, degi0, degi1)

  part = _make_agg_call()(x_pad, src2, dst2, zrows)

  W2p = jnp.concatenate([W2, jnp.zeros((H, CPAD - C), f32)], axis=1)
  b2p = jnp.concatenate([b2, jnp.full((CPAD - C,), -1e30, f32)]).reshape(1, CPAD)
  b1r = b1.reshape(1, H)

  RD = 640
  gridd = NPAD // RD
  out = pl.pallas_call(
      _mlp_body,
      grid=(gridd,),
      in_specs=[
          pl.BlockSpec((RD, D), lambda i: (i, 0)),
          pl.BlockSpec((RD, D), lambda i: (i + NPAD // RD, 0)),
          pl.BlockSpec((RD, 1), lambda i: (i, 0)),
          pl.BlockSpec((D, H), lambda i: (0, 0)),
          pl.BlockSpec((1, H), lambda i: (0, 0)),
          pl.BlockSpec((H, CPAD), lambda i: (0, 0)),
          pl.BlockSpec((1, CPAD), lambda i: (0, 0)),
      ],
      out_specs=pl.BlockSpec((RD, CPAD), lambda i: (i, 0)),
      out_shape=jax.ShapeDtypeStruct((NPAD, CPAD), f32),
  )(part, part, norm_in, W1, b1r, W2p, b2p)

  return out[:N, :C]
